# baseline (speedup 1.0000x reference)
import jax
import jax.numpy as jnp
from jax import lax
from jax.experimental import pallas as pl
from jax.experimental.pallas import tpu as pltpu

IDIM = 80
ODIM = 80
HDIM = 512
CDIM = 128
TEMPER = 1.0
IGNORE_OUT = 0.0
N_ITERS = 4

TB = 1024
NSH = 2 * ODIM - 1
TPAD = 240


def _uproll(x, s):
    r = x.shape[0]
    s = s % r
    if s == 0:
        return x
    return pltpu.roll(x, r - s, 0)


def _shift_rows_by_lane(v, s):
    for k in range(8):
        bit = ((s >> k) & 1) > 0
        v = jnp.where(bit, _uproll(v, 1 << k), v)
    return v


def _count_ge(r, cand):
    m = (r >= cand).astype(jnp.float32)
    return jnp.sum(m, axis=0, keepdims=True)


def _topk_taus(rs):
    k = CDIM - 0.5

    def body(j, taus):
        b = 30 - j
        step = jnp.int32(1) << b
        out = []
        for r, tau in zip(rs, taus):
            cand = tau + step
            out.append(jnp.where(_count_ge(r, cand) >= k, cand, tau))
        return tuple(out)

    taus = tuple(jnp.zeros((1, r.shape[1]), jnp.int32) for r in rs)
    return lax.fori_loop(0, 31, body, taus)


def _topk_mask(e):
    r = lax.bitcast_convert_type(e, jnp.int32)
    (tau,) = _topk_taus((r,))
    return (r >= tau).astype(jnp.float32)


def _topk_mask2(e1, e2):
    r1 = lax.bitcast_convert_type(e1, jnp.int32)
    r2 = lax.bitcast_convert_type(e2, jnp.int32)
    tau1, tau2 = _topk_taus((r1, r2))
    return (r1 >= tau1).astype(jnp.float32), (r2 >= tau2).astype(jnp.float32)


def _body(x_ref, y_ref, we_ref, be_ref, wd_ref, bd_ref, out_ref):
    i = pl.program_id(0)
    y0 = y_ref[...]
    we = we_ref[...]
    be = be_ref[...]
    wd = wd_ref[...]
    bd = bd_ref[...]

    seq_mask = y0 == IGNORE_OUT
    sm_tokT = jnp.min(
        jnp.where(jnp.swapaxes(y0, 0, 1) == IGNORE_OUT, 1.0, 0.0),
        axis=0, keepdims=True,
    )
    sub = lax.broadcasted_iota(jnp.int32, (NSH + 1, TB), 0)
    ones_o = jnp.ones((ODIM, 1), jnp.float32)

    x_resT = jnp.swapaxes(x_ref[...], 0, 1)
    y_res = y0
    mask_prev = jnp.zeros((HDIM, TB), jnp.float32)
    total = jnp.float32(0.0)

    for it in range(N_ITERS):
        y_resT = jnp.swapaxes(y_res, 0, 1)
        x_padT = jnp.pad(x_resT, ((ODIM - 1, TPAD - (IDIM + ODIM - 1)), (0, 0)))

        num = jnp.zeros((NSH + 1, TB), jnp.float32)
        for b in range(8):
            xb = _uproll(x_padT, b)
            for a in range(ODIM // 8):
                w = 8 * a + b
                ybc = y_resT[w : w + 1, :]
                num = num + xb[8 * a : 8 * a + NSH + 1] * ybc

        xsq = x_padT * x_padT
        s = xsq + _uproll(xsq, 1)
        s = s + _uproll(s, 2)
        s = s + _uproll(s, 4)
        w16 = s + _uproll(s, 8)
        s = w16 + _uproll(w16, 16)
        s = s + _uproll(s, 32)
        nsq = (s + _uproll(w16, 64))[: NSH + 1]

        ynormT = jnp.sqrt(jnp.sum(y_resT * y_resT, axis=0, keepdims=True))
        den = jnp.sqrt(nsq) * ynormT + 1e-8
        simT = jnp.where(sub < NSH, num / den, -jnp.inf)
        mT = jnp.max(simT, axis=0, keepdims=True)
        thetaT = jnp.min(jnp.where(simT == mT, sub, TPAD), axis=0, keepdims=True)

        y_alignT = _shift_rows_by_lane(x_padT, thetaT)[:ODIM]

        zT = y_alignT * y_resT * (1.0 / TEMPER)
        ezT = jnp.exp(zT - jnp.max(zT, axis=0, keepdims=True))
        attnT = ezT / jnp.sum(ezT, axis=0, keepdims=True)
        v2T = y_alignT * attnT

        v2pT = jnp.pad(v2T, ((ODIM - 1, TPAD - (2 * ODIM - 1)), (0, 0)))
        x_eleT = _shift_rows_by_lane(v2pT, 2 * ODIM - 2 - thetaT)[:IDIM]

        h = lax.dot_general(
            we, v2T, (((0,), (0,)), ((), ())),
            preferred_element_type=jnp.float32,
        ) + be

        if it == 0:
            mask_cur = _topk_mask(h * h)
            mask_prev = mask_cur
            h2 = h * mask_cur
            loss_h = jnp.float32(0.0)
        else:
            hz = jnp.where(mask_prev > 0, 0.0, h)
            mc1, mc2 = _topk_mask2(h * h, hz * hz)
            mask_int = mask_prev * mc1
            lh = (h - (1.0 - mask_int)) ** 2
            lh = jnp.where(sm_tokT > 0, 0.0, lh)
            lh = jnp.where(mask_int > 0, lh, 0.0)
            loss_h = jnp.sum(lh)
            mask_prev = mask_prev + mc2
            h2 = hz * mc2

        y_ele = lax.dot_general(
            h2, wd, (((0,), (0,)), ((), ())),
            preferred_element_type=jnp.float32,
        ) + bd

        inv_me = 1.0 / (jnp.abs(thetaT.astype(jnp.float32) - (ODIM - 1.0)) + 1.0)
        llm = jnp.where(seq_mask, 0.0, (y_ele - y_res) ** 2)
        rowsum = jnp.dot(llm, ones_o, preferred_element_type=jnp.float32)
        ll_tot = jnp.dot(inv_me, rowsum, preferred_element_type=jnp.float32)
        total = total + ll_tot[0, 0] + loss_h

        y_res = y_res - y_ele
        x_resT = x_resT - x_eleT

    @pl.when(i == 0)
    def _init():
        out_ref[0, 0] = jnp.float32(0.0)

    out_ref[0, 0] += total / N_ITERS


@jax.jit
def kernel(x, y, W_enc, b_enc, W_dec, b_dec):
    n = x.shape[0] * x.shape[1]
    xr = x.reshape(n, IDIM)
    yr = y.reshape(n, ODIM)
    grid = n // TB
    out = pl.pallas_call(
        _body,
        grid=(grid,),
        in_specs=[
            pl.BlockSpec((TB, IDIM), lambda i: (i, 0)),
            pl.BlockSpec((TB, ODIM), lambda i: (i, 0)),
            pl.BlockSpec((IDIM, HDIM), lambda i: (0, 0)),
            pl.BlockSpec((HDIM, 1), lambda i: (0, 0)),
            pl.BlockSpec((HDIM, ODIM), lambda i: (0, 0)),
            pl.BlockSpec((1, ODIM), lambda i: (0, 0)),
        ],
        out_specs=pl.BlockSpec(
            (1, 1), lambda i: (0, 0), memory_space=pltpu.SMEM
        ),
        out_shape=jax.ShapeDtypeStruct((1, 1), jnp.float32),
        interpret=False,
    )(xr, yr, W_enc, b_enc.reshape(HDIM, 1), W_dec, b_dec.reshape(1, ODIM))
    return out[0, 0]
